# Initial kernel scaffold; baseline (speedup 1.0000x reference)
import jax
import jax.numpy as jnp
from jax.experimental import pallas as pl


def kernel(x, weight, shift, post_gain):
    raise NotImplementedError("write your pallas kernel here")


# speedup vs baseline: 1.0712x; 1.0712x over previous
import functools

import jax
import jax.numpy as jnp
from jax.experimental import pallas as pl
from jax.experimental.pallas import tpu as pltpu

S = 1.0
EPS = 1e-12
SINH_CLIP = 30.0
VAR_FLOOR = 1e-3
BN_EPS = 1e-6
MAX_STEP = 0.5
TOL = 1e-6
N_ITERS = 8

_P = 1
_FMAX = 3.4028235e38


def _asinh(n):
    return jnp.log(n + jnp.sqrt(n * n + 1.0))


def _sinh(a):
    e = jnp.exp(a)
    return 0.5 * (e - 1.0 / e)


def _nan_to_num(v):
    v = jnp.where(jnp.isnan(v), jnp.float32(0.0), v)
    return jnp.clip(v, -_FMAX, _FMAX)


def _rownorm2(v):
    return jnp.sum(v * v, axis=1, keepdims=True)


def _center(xb, m):
    mm = jnp.sum(m * m)
    bu = jax.lax.rsqrt(1.0 + mm)
    t = jnp.sum(xb * m[None, :], axis=1, keepdims=True)
    bv = jax.lax.rsqrt(1.0 + _rownorm2(xb))
    coef = (bu / (1.0 + bu)) * (-t) + (1.0 - bv) / bv
    return xb - (1.0 + coef) * m[None, :]


def _step_kernel(x_ref, aux_ref, out_ref):
    b = pl.program_id(1)
    m = aux_ref[0, :]
    y = _center(x_ref[...], m)
    n2 = _rownorm2(y)
    n = jnp.sqrt(n2)
    g = _asinh(n) / jnp.maximum(n, EPS)
    g = jnp.where(n <= EPS, jnp.float32(0.0), g)
    contrib = jnp.sum(g * y, axis=0, keepdims=True)[None]

    @pl.when(b == 0)
    def _():
        out_ref[...] = contrib

    @pl.when(b != 0)
    def _():
        out_ref[...] += contrib


def _var_kernel(x_ref, aux_ref, out_ref):
    b = pl.program_id(1)
    m = aux_ref[0, :]
    y = _center(x_ref[...], m)
    n = jnp.sqrt(_rownorm2(y))
    d = _asinh(n)
    s = jnp.sum(d * d)
    contrib = jnp.full((1, 1, 128), s, dtype=jnp.float32)

    @pl.when(b == 0)
    def _():
        out_ref[...] = contrib

    @pl.when(b != 0)
    def _():
        out_ref[...] += contrib


def _xform_kernel(x_ref, aux_ref, out_ref):
    m = aux_ref[0, :]
    w = aux_ref[1, :]
    factor = aux_ref[2, 0]
    gain = aux_ref[3, 0]

    y = _nan_to_num(_center(x_ref[...], m))

    n = jnp.sqrt(_rownorm2(y))
    ra = jnp.clip(factor * _asinh(n), -SINH_CLIP, SINH_CLIP)
    c1 = _sinh(ra) / jnp.maximum(n, EPS)
    z = jnp.where(n <= EPS, jnp.float32(0.0), c1 * y)

    ww = jnp.sum(w * w)
    bu = jax.lax.rsqrt(1.0 + ww)
    bv = jax.lax.rsqrt(1.0 + _rownorm2(z))
    coef = (bu / (1.0 + bu)) * jnp.sum(w[None, :] * z, axis=1, keepdims=True) \
        + (1.0 - bv) / bv
    x1 = w[None, :] + z + coef * w[None, :]

    x1 = _nan_to_num(x1)
    n1 = jnp.sqrt(_rownorm2(x1))
    ra1 = jnp.clip(gain * _asinh(n1), -SINH_CLIP, SINH_CLIP)
    c2 = _sinh(ra1) / jnp.maximum(n1, EPS)
    out_ref[...] = jnp.where(n1 <= EPS, jnp.float32(0.0), c2 * x1)


def _expmap0_vec(v):
    n = jnp.linalg.norm(v, axis=-1, keepdims=True)
    coef = jnp.sinh(n) / jnp.maximum(n, EPS)
    return jnp.where(n <= EPS, jnp.zeros_like(v), coef * v)


def _gyro_add_vec(u, v):
    bu = jax.lax.rsqrt(1.0 + jnp.sum(u * u, axis=-1, keepdims=True))
    bv = jax.lax.rsqrt(1.0 + jnp.sum(v * v, axis=-1, keepdims=True))
    coef = (bu / (1.0 + bu)) * jnp.sum(u * v, axis=-1, keepdims=True) \
        + (1.0 - bv) / bv
    return u + v + coef * u


def _make_aux(mean, w_pt=None, factor=None, gain=None):
    aux = jnp.zeros((8, 128), dtype=jnp.float32)
    aux = aux.at[0, :].set(mean)
    if w_pt is not None:
        aux = aux.at[1, :].set(w_pt)
        aux = aux.at[2, 0].set(factor)
        aux = aux.at[3, 0].set(gain)
    return aux


@jax.jit
def kernel(x, weight, shift, post_gain):
    orig_shape = x.shape
    xf = x.reshape(-1, x.shape[-1]).astype(jnp.float32)
    n_rows, d = xf.shape

    blk = 2048 if n_rows % (_P * 2048) == 0 else n_rows // _P
    nb = n_rows // (_P * blk)
    grid = (_P, nb)

    x_spec = pl.BlockSpec((blk, d), lambda p, b: (p * nb + b, 0))
    aux_spec = pl.BlockSpec((8, 128), lambda p, b: (0, 0))
    acc_spec = pl.BlockSpec((1, 1, d), lambda p, b: (p, 0, 0))
    params = pltpu.CompilerParams(
        dimension_semantics=(
            pltpu.GridDimensionSemantics.PARALLEL,
            pltpu.GridDimensionSemantics.ARBITRARY,
        ),
    )

    step_call = pl.pallas_call(
        _step_kernel,
        grid=grid,
        in_specs=[x_spec, aux_spec],
        out_specs=acc_spec,
        out_shape=jax.ShapeDtypeStruct((_P, 1, d), jnp.float32),
        compiler_params=params,
    )
    var_call = pl.pallas_call(
        _var_kernel,
        grid=grid,
        in_specs=[x_spec, aux_spec],
        out_specs=acc_spec,
        out_shape=jax.ShapeDtypeStruct((_P, 1, 128), jnp.float32),
        compiler_params=params,
    )
    xform_call = pl.pallas_call(
        _xform_kernel,
        grid=grid,
        in_specs=[x_spec, aux_spec],
        out_specs=x_spec,
        out_shape=jax.ShapeDtypeStruct((n_rows, d), jnp.float32),
        compiler_params=params,
    )

    mean = xf[0:1]
    done = jnp.asarray(False)
    for _ in range(N_ITERS):
        psum = step_call(xf, _make_aux(mean[0]))[:, 0]
        step = jnp.sum(psum, axis=0, keepdims=True) / n_rows
        sn = jnp.maximum(jnp.linalg.norm(step), 1e-8)
        step = step * jnp.minimum(MAX_STEP / sn, 1.0)
        new_mean = _gyro_add_vec(mean, _expmap0_vec(step))
        conv = jnp.linalg.norm(new_mean - mean) < TOL
        mean = jnp.where(done, mean, new_mean)
        done = jnp.logical_or(done, conv)
    mean_v = mean[0]

    vpart = var_call(xf, _make_aux(mean_v))
    var = jnp.maximum(_nan_to_num(jnp.sum(vpart[:, 0, 0]) / n_rows), 1e-8)

    w_pt = _expmap0_vec(weight[None, :])[0]
    factor = (shift / jnp.sqrt(jnp.maximum(var, VAR_FLOOR) + BN_EPS))[0]
    gain = jnp.clip(post_gain, 0.5, 3.0)
    out = xform_call(xf, _make_aux(mean_v, w_pt, factor, gain))
    return out.reshape(orig_shape)
